# Initial kernel scaffold; baseline (speedup 1.0000x reference)
import jax
import jax.numpy as jnp
from jax.experimental import pallas as pl


def kernel(image_features, text_features, W_img, W_txt, W_final):
    raise NotImplementedError("write your pallas kernel here")


# speedup vs baseline: 2.0807x; 2.0807x over previous
import functools

import jax
import jax.numpy as jnp
from jax import lax
from jax.experimental import pallas as pl
from jax.experimental.pallas import tpu as pltpu
from jax.experimental.pallas import tpu_sc as plsc

B = 1024
N_TEXT = 16384
D = 1024
K = 8

_BF = jnp.bfloat16


def _mm(a, b, dims):
    return lax.dot_general(a.astype(_BF), b.astype(_BF), (dims, ((), ())),
                           preferred_element_type=jnp.float32)


def _proj_body(img_ref, wimg_ref, ip_ref):
    ip_ref[...] = _mm(img_ref[...], wimg_ref[...], ((1,), (1,)))


def _proj(image_features, W_img):
    return pl.pallas_call(
        _proj_body,
        out_shape=jax.ShapeDtypeStruct((B, D), jnp.float32),
    )(image_features, W_img)


B_BLK = 256
N_BLK = 2048
NI = B // B_BLK
NJ = N_TEXT // N_BLK


def _topk_body(text_ref, wtxt_ref, ip_ref, tp_ref, w_ref, idx_ref,
               rv_ref, ri_ref):
    j = pl.program_id(0)
    i = pl.program_id(1)
    rows = pl.ds(i * B_BLK, B_BLK)

    @pl.when(i == 0)
    def _tp():
        tp_ref[...] = _mm(text_ref[...], wtxt_ref[...], ((1,), (1,)))

    @pl.when(j == 0)
    def _init():
        rv_ref[rows, :] = jnp.full((B_BLK, K), -jnp.inf, jnp.float32)
        ri_ref[rows, :] = jnp.zeros((B_BLK, K), jnp.int32)

    s = _mm(ip_ref[...], tp_ref[...], ((1,), (1,)))

    col = lax.broadcasted_iota(jnp.int32, (B_BLK, N_BLK), 1)
    k8 = lax.broadcasted_iota(jnp.int32, (B_BLK, K), 1)
    bv = jnp.full((B_BLK, K), -jnp.inf, jnp.float32)
    bi = jnp.zeros((B_BLK, K), jnp.int32)
    for t in range(K):
        m = jnp.max(s, axis=1)
        hit = s >= m[:, None]
        a = jnp.min(jnp.where(hit, col, N_TEXT), axis=1)
        bv = jnp.where(k8 == t, m[:, None], bv)
        bi = jnp.where(k8 == t, a[:, None] + j * N_BLK, bi)
        s = jnp.where(col == a[:, None], -jnp.inf, s)

    mv = jnp.concatenate([rv_ref[rows, :], bv], axis=1)
    mi = jnp.concatenate([ri_ref[rows, :], bi], axis=1)
    col16 = lax.broadcasted_iota(jnp.int32, (B_BLK, 2 * K), 1)
    nv = jnp.zeros((B_BLK, K), jnp.float32)
    ni = jnp.zeros((B_BLK, K), jnp.int32)
    for t in range(K):
        m = jnp.max(mv, axis=1)
        sel = (mv >= m[:, None]) & (
            col16 <= jnp.min(jnp.where(mv >= m[:, None], col16, 2 * K),
                             axis=1)[:, None])
        a = jnp.sum(jnp.where(sel, mi, 0), axis=1)
        nv = jnp.where(k8 == t, m[:, None], nv)
        ni = jnp.where(k8 == t, a[:, None], ni)
        mv = jnp.where(sel, -jnp.inf, mv)
    rv_ref[rows, :] = nv
    ri_ref[rows, :] = ni

    @pl.when(j == NJ - 1)
    def _emit():
        v = nv
        e = jnp.exp(v - jnp.max(v, axis=1, keepdims=True))
        w = e / jnp.sum(e, axis=1, keepdims=True)
        w_ref[...] = jnp.broadcast_to(w[:, :, None], (B_BLK, K, 16))
        idx_ref[...] = ni


def _topk(img_proj, text_features, W_txt):
    return pl.pallas_call(
        _topk_body,
        grid=(NJ, NI),
        in_specs=[
            pl.BlockSpec((N_BLK, D), lambda j, i: (j, 0)),
            pl.BlockSpec((D, D), lambda j, i: (0, 0)),
            pl.BlockSpec((B_BLK, D), lambda j, i: (i, 0)),
        ],
        out_specs=[
            pl.BlockSpec((N_BLK, D), lambda j, i: (j, 0)),
            pl.BlockSpec((B_BLK, K, 16), lambda j, i: (i, 0, 0)),
            pl.BlockSpec((B_BLK, K), lambda j, i: (i, 0)),
        ],
        out_shape=[
            jax.ShapeDtypeStruct((N_TEXT, D), jnp.float32),
            jax.ShapeDtypeStruct((B, K, 16), jnp.float32),
            jax.ShapeDtypeStruct((B, K), jnp.int32),
        ],
        scratch_shapes=[
            pltpu.VMEM((B, K), jnp.float32),
            pltpu.VMEM((B, K), jnp.int32),
        ],
        compiler_params=pltpu.CompilerParams(
            dimension_semantics=("arbitrary", "arbitrary")),
    )(text_features, W_txt, img_proj)


_NC = 2
_NS = 16
_NW = _NC * _NS
RPW = B // _NW


def _gather_body(idx_hbm, w_hbm, tp_hbm, out_hbm, idx_v, wrow_v, rows_v,
                 acc_v, sem):
    wid = lax.axis_index("s") * _NC + lax.axis_index("c")
    base = wid * RPW
    pltpu.sync_copy(idx_hbm.at[pl.ds(base, RPW)], idx_v)

    def row(r, carry):
        pltpu.async_copy(tp_hbm.at[idx_v.at[r]], rows_v, sem).wait()
        pltpu.sync_copy(w_hbm.at[base + r], wrow_v)
        wk = [wrow_v[k] for k in range(K)]
        for c in range(D // 16):
            acc = wk[0] * rows_v[0, pl.ds(c * 16, 16)]
            for k in range(1, K):
                acc = acc + wk[k] * rows_v[k, pl.ds(c * 16, 16)]
            acc_v[pl.ds(c * 16, 16)] = acc
        pltpu.sync_copy(acc_v, out_hbm.at[base + r])
        return carry

    lax.fori_loop(0, RPW, row, 0)


def _sc_gather(top_idx, top_w, txt_proj):
    mesh = plsc.VectorSubcoreMesh(core_axis_name="c", subcore_axis_name="s")
    fn = functools.partial(
        pl.kernel,
        mesh=mesh,
        out_type=jax.ShapeDtypeStruct((B, D), jnp.float32),
        scratch_types=[
            pltpu.VMEM((RPW, K), jnp.int32),
            pltpu.VMEM((K, 16), jnp.float32),
            pltpu.VMEM((K, D), jnp.float32),
            pltpu.VMEM((D,), jnp.float32),
            pltpu.SemaphoreType.DMA,
        ],
    )(_gather_body)
    return fn(top_idx, top_w, txt_proj)


def _final_body(ip_ref, g_ref, wfinal_ref, out_ref):
    enh = ip_ref[...] * g_ref[...]
    out = _mm(enh, wfinal_ref[...], ((1,), (1,)))
    out_ref[...] = jnp.maximum(out, 0.0)


def _final(img_proj, g, W_final):
    return pl.pallas_call(
        _final_body,
        out_shape=jax.ShapeDtypeStruct((B, D), jnp.float32),
    )(img_proj, g, W_final)


def kernel(image_features, text_features, W_img, W_txt, W_final):
    img_proj = _proj(image_features, W_img)
    txt_proj, top_w, top_idx = _topk(img_proj, text_features, W_txt)
    g = _sc_gather(top_idx, top_w, txt_proj)
    return _final(img_proj, g, W_final)


# speedup vs baseline: 2.2717x; 1.0918x over previous
import functools

import jax
import jax.numpy as jnp
from jax import lax
from jax.experimental import pallas as pl
from jax.experimental.pallas import tpu as pltpu
from jax.experimental.pallas import tpu_sc as plsc

B = 1024
N_TEXT = 16384
D = 1024
K = 8

_BF = jnp.bfloat16


def _mm(a, b, dims):
    return lax.dot_general(a.astype(_BF), b.astype(_BF), (dims, ((), ())),
                           preferred_element_type=jnp.float32)


def _proj_body(img_ref, wimg_ref, ip_ref):
    ip_ref[...] = _mm(img_ref[...], wimg_ref[...], ((1,), (1,)))


def _proj(image_features, W_img):
    return pl.pallas_call(
        _proj_body,
        out_shape=jax.ShapeDtypeStruct((B, D), jnp.float32),
    )(image_features, W_img)


B_BLK = 256
N_BLK = 2048
NI = B // B_BLK
NJ = N_TEXT // N_BLK


def _topk_body(text_ref, wtxt_ref, ip_ref, tp_ref, w_ref, idx_ref,
               cv_ref, ci_ref, lb_ref, s_ref, m_ref):
    j = pl.program_id(0)
    i = pl.program_id(1)
    rows = pl.ds(i * B_BLK, B_BLK)

    @pl.when(i == 0)
    def _tp():
        tp_ref[...] = _mm(text_ref[...], wtxt_ref[...], ((1,), (1,)))

    @pl.when(j == 0)
    def _init():
        lb_ref[rows, :] = jnp.full((B_BLK, 1), -jnp.inf, jnp.float32)

    s0 = _mm(ip_ref[...], tp_ref[...], ((1,), (1,)))
    s_ref[...] = s0
    m_ref[...] = jnp.max(s0, axis=1, keepdims=True)

    colf = lax.broadcasted_iota(jnp.int32, (B_BLK, N_BLK), 1).astype(
        jnp.float32)
    k8 = lax.broadcasted_iota(jnp.int32, (B_BLK, K), 1)
    rv_min = lb_ref[rows, :]

    col128 = lax.broadcasted_iota(jnp.int32, (B_BLK, 128), 1)
    in_slot = (col128 >= j * K) & (col128 < j * K + K)
    cur_v = jnp.where(j == 0, jnp.full((B_BLK, 128), -jnp.inf, jnp.float32),
                      cv_ref[rows, :])
    cv_ref[rows, :] = jnp.where(in_slot, -jnp.inf, cur_v)

    for t in range(K):
        @pl.when(jnp.any(m_ref[...] > rv_min))
        def _iter(t=t):
            s = s_ref[...]
            m = m_ref[...][:, 0]
            hit = s >= m[:, None]
            af = jnp.min(jnp.where(hit, colf, jnp.float32(N_BLK)), axis=1)
            gidx = af.astype(jnp.int32) + j * N_BLK
            lane = col128 == (j * K + t)
            cv_ref[rows, :] = jnp.where(lane, m[:, None], cv_ref[rows, :])
            ci_ref[rows, :] = jnp.where(lane, gidx[:, None], ci_ref[rows, :])
            s2 = jnp.where(hit, -jnp.inf, s)
            s_ref[...] = s2
            m_ref[...] = jnp.max(s2, axis=1, keepdims=True)
            if t == K - 1:
                lb_ref[rows, :] = jnp.maximum(rv_min, m[:, None])

    @pl.when(j == NJ - 1)
    def _emit():
        cv = cv_ref[rows, :]
        ci = ci_ref[rows, :]
        colc = lax.broadcasted_iota(jnp.int32, (B_BLK, 128), 1).astype(
            jnp.float32)
        nv = jnp.zeros((B_BLK, K), jnp.float32)
        ni = jnp.zeros((B_BLK, K), jnp.int32)
        for t in range(K):
            m = jnp.max(cv, axis=1)
            hit = cv >= m[:, None]
            af = jnp.min(jnp.where(hit, colc, jnp.float32(128)), axis=1)
            a = jnp.sum(jnp.where(colc == af[:, None], ci, 0), axis=1)
            nv = jnp.where(k8 == t, m[:, None], nv)
            ni = jnp.where(k8 == t, a[:, None], ni)
            cv = jnp.where(hit, -jnp.inf, cv)
        e = jnp.exp(nv - nv[:, 0:1])
        w = e / jnp.sum(e, axis=1, keepdims=True)
        w_ref[...] = jnp.broadcast_to(w[:, :, None], (B_BLK, K, 16))
        idx_ref[...] = ni


def _topk(img_proj, text_features, W_txt):
    return pl.pallas_call(
        _topk_body,
        grid=(NJ, NI),
        in_specs=[
            pl.BlockSpec((N_BLK, D), lambda j, i: (j, 0)),
            pl.BlockSpec((D, D), lambda j, i: (0, 0)),
            pl.BlockSpec((B_BLK, D), lambda j, i: (i, 0)),
        ],
        out_specs=[
            pl.BlockSpec((N_BLK, D), lambda j, i: (j, 0)),
            pl.BlockSpec((B_BLK, K, 16), lambda j, i: (i, 0, 0)),
            pl.BlockSpec((B_BLK, K), lambda j, i: (i, 0)),
        ],
        out_shape=[
            jax.ShapeDtypeStruct((N_TEXT, D), jnp.float32),
            jax.ShapeDtypeStruct((B, K, 16), jnp.float32),
            jax.ShapeDtypeStruct((B, K), jnp.int32),
        ],
        scratch_shapes=[
            pltpu.VMEM((B, 128), jnp.float32),
            pltpu.VMEM((B, 128), jnp.int32),
            pltpu.VMEM((B, 1), jnp.float32),
            pltpu.VMEM((B_BLK, N_BLK), jnp.float32),
            pltpu.VMEM((B_BLK, 1), jnp.float32),
        ],
        compiler_params=pltpu.CompilerParams(
            dimension_semantics=("arbitrary", "arbitrary")),
    )(text_features, W_txt, img_proj)


_NC = 2
_NS = 16
_NW = _NC * _NS
RPW = B // _NW


def _gather_body(idx_hbm, w_hbm, tp_hbm, out_hbm, idx_v, wrow_v, rows_v,
                 acc_v, sem):
    wid = lax.axis_index("s") * _NC + lax.axis_index("c")
    base = wid * RPW
    pltpu.sync_copy(idx_hbm.at[pl.ds(base, RPW)], idx_v)

    def row(r, carry):
        pltpu.async_copy(tp_hbm.at[idx_v.at[r]], rows_v, sem).wait()
        pltpu.sync_copy(w_hbm.at[base + r], wrow_v)
        wk = [wrow_v[k] for k in range(K)]
        for c in range(D // 16):
            acc = wk[0] * rows_v[0, pl.ds(c * 16, 16)]
            for k in range(1, K):
                acc = acc + wk[k] * rows_v[k, pl.ds(c * 16, 16)]
            acc_v[pl.ds(c * 16, 16)] = acc
        pltpu.sync_copy(acc_v, out_hbm.at[base + r])
        return carry

    lax.fori_loop(0, RPW, row, 0)


def _sc_gather(top_idx, top_w, txt_proj):
    mesh = plsc.VectorSubcoreMesh(core_axis_name="c", subcore_axis_name="s")
    fn = functools.partial(
        pl.kernel,
        mesh=mesh,
        out_type=jax.ShapeDtypeStruct((B, D), jnp.float32),
        scratch_types=[
            pltpu.VMEM((RPW, K), jnp.int32),
            pltpu.VMEM((K, 16), jnp.float32),
            pltpu.VMEM((K, D), jnp.float32),
            pltpu.VMEM((D,), jnp.float32),
            pltpu.SemaphoreType.DMA,
        ],
    )(_gather_body)
    return fn(top_idx, top_w, txt_proj)


def _final_body(ip_ref, g_ref, wfinal_ref, out_ref):
    enh = ip_ref[...] * g_ref[...]
    out = _mm(enh, wfinal_ref[...], ((1,), (1,)))
    out_ref[...] = jnp.maximum(out, 0.0)


def _final(img_proj, g, W_final):
    return pl.pallas_call(
        _final_body,
        out_shape=jax.ShapeDtypeStruct((B, D), jnp.float32),
    )(img_proj, g, W_final)


def kernel(image_features, text_features, W_img, W_txt, W_final):
    img_proj = _proj(image_features, W_img)
    txt_proj, top_w, top_idx = _topk(img_proj, text_features, W_txt)
    g = _sc_gather(top_idx, top_w, txt_proj)
    return _final(img_proj, g, W_final)


# speedup vs baseline: 2.9805x; 1.3120x over previous
import functools

import jax
import jax.numpy as jnp
from jax import lax
from jax.experimental import pallas as pl
from jax.experimental.pallas import tpu as pltpu
from jax.experimental.pallas import tpu_sc as plsc

B = 1024
N_TEXT = 16384
D = 1024
K = 8

_BF = jnp.bfloat16


def _mm(a, b, dims):
    return lax.dot_general(a.astype(_BF), b.astype(_BF), (dims, ((), ())),
                           preferred_element_type=jnp.float32)


def _proj_body(img_ref, wimg_ref, ip_ref):
    ip_ref[...] = _mm(img_ref[...], wimg_ref[...], ((1,), (1,)))


def _proj(image_features, W_img):
    return pl.pallas_call(
        _proj_body,
        out_shape=jax.ShapeDtypeStruct((B, D), jnp.float32),
    )(image_features, W_img)


B_BLK = 256
N_BLK = 2048
NI = B // B_BLK
NJ = N_TEXT // N_BLK


def _topk_body(text_ref, wtxt_ref, ip_ref, tp_ref, w_ref, idx_ref,
               cv_ref, ci_ref):
    j = pl.program_id(0)
    i = pl.program_id(1)
    rows = pl.ds(i * B_BLK, B_BLK)

    @pl.when(i == 0)
    def _tp():
        tp_ref[...] = _mm(text_ref[...], wtxt_ref[...], ((1,), (1,)))

    s = _mm(ip_ref[...], tp_ref[...], ((1,), (1,)))

    colf = lax.broadcasted_iota(jnp.int32, (B_BLK, N_BLK), 1).astype(
        jnp.float32)
    k8 = lax.broadcasted_iota(jnp.int32, (B_BLK, K), 1)

    bv = jnp.full((B_BLK, K), -jnp.inf, jnp.float32)
    bi = jnp.zeros((B_BLK, K), jnp.int32)
    for t in range(K):
        m = jnp.max(s, axis=1)
        hit = s >= m[:, None]
        af = jnp.min(jnp.where(hit, colf, jnp.float32(N_BLK)), axis=1)
        bv = jnp.where(k8 == t, m[:, None], bv)
        bi = jnp.where(k8 == t, af.astype(jnp.int32)[:, None] + j * N_BLK,
                       bi)
        s = jnp.where(hit, -jnp.inf, s)

    col128 = lax.broadcasted_iota(jnp.int32, (B_BLK, 128), 1)
    in_slot = (col128 >= j * K) & (col128 < j * K + K)
    cur_v = jnp.where(j == 0, jnp.full((B_BLK, 128), -jnp.inf, jnp.float32),
                      cv_ref[rows, :])
    cur_i = jnp.where(j == 0, jnp.zeros((B_BLK, 128), jnp.int32),
                      ci_ref[rows, :])
    cv_ref[rows, :] = jnp.where(in_slot, jnp.tile(bv, (1, 128 // K)), cur_v)
    ci_ref[rows, :] = jnp.where(in_slot, jnp.tile(bi, (1, 128 // K)), cur_i)

    @pl.when(j == NJ - 1)
    def _emit():
        cv = cv_ref[rows, :]
        ci = ci_ref[rows, :]
        colc = lax.broadcasted_iota(jnp.int32, (B_BLK, 128), 1).astype(
            jnp.float32)
        nv = jnp.zeros((B_BLK, K), jnp.float32)
        ni = jnp.zeros((B_BLK, K), jnp.int32)
        for t in range(K):
            m = jnp.max(cv, axis=1)
            hit = cv >= m[:, None]
            af = jnp.min(jnp.where(hit, colc, jnp.float32(128)), axis=1)
            a = jnp.sum(jnp.where(colc == af[:, None], ci, 0), axis=1)
            nv = jnp.where(k8 == t, m[:, None], nv)
            ni = jnp.where(k8 == t, a[:, None], ni)
            cv = jnp.where(hit, -jnp.inf, cv)
        e = jnp.exp(nv - nv[:, 0:1])
        w = e / jnp.sum(e, axis=1, keepdims=True)
        w_ref[...] = jnp.broadcast_to(w[:, :, None], (B_BLK, K, 16))
        idx_ref[...] = ni


def _topk(img_proj, text_features, W_txt):
    return pl.pallas_call(
        _topk_body,
        grid=(NJ, NI),
        in_specs=[
            pl.BlockSpec((N_BLK, D), lambda j, i: (j, 0)),
            pl.BlockSpec((D, D), lambda j, i: (0, 0)),
            pl.BlockSpec((B_BLK, D), lambda j, i: (i, 0)),
        ],
        out_specs=[
            pl.BlockSpec((N_BLK, D), lambda j, i: (j, 0)),
            pl.BlockSpec((B_BLK, K, 16), lambda j, i: (i, 0, 0)),
            pl.BlockSpec((B_BLK, K), lambda j, i: (i, 0)),
        ],
        out_shape=[
            jax.ShapeDtypeStruct((N_TEXT, D), jnp.float32),
            jax.ShapeDtypeStruct((B, K, 16), jnp.float32),
            jax.ShapeDtypeStruct((B, K), jnp.int32),
        ],
        scratch_shapes=[
            pltpu.VMEM((B, 128), jnp.float32),
            pltpu.VMEM((B, 128), jnp.int32),
        ],
        compiler_params=pltpu.CompilerParams(
            dimension_semantics=("arbitrary", "arbitrary")),
    )(text_features, W_txt, img_proj)


_NC = 2
_NS = 16
_NW = _NC * _NS
RPW = B // _NW


G = 2
NB = RPW // G


def _gather_body(idx_hbm, w_hbm, tp_hbm, out_hbm, idx_v, wall_v, rows_v,
                 acc_v, sg0, sg1):
    wid = lax.axis_index("s") * _NC + lax.axis_index("c")
    base = wid * RPW
    pltpu.sync_copy(idx_hbm.at[pl.ds(base * K, RPW * K)], idx_v)
    pltpu.sync_copy(w_hbm.at[pl.ds(base * K, RPW * K)], wall_v)
    pltpu.async_copy(tp_hbm.at[idx_v.at[pl.ds(0, G * K)]], rows_v.at[0], sg0)

    def accum(g, b):
        for rr in range(G):
            r = g * G + rr

            def chunk(c, wk):
                acc = wk[0] * rows_v[b, rr * K, pl.ds(c * 16, 16)]
                for k in range(1, K):
                    acc = acc + wk[k] * rows_v[b, rr * K + k,
                                               pl.ds(c * 16, 16)]
                acc_v[r, pl.ds(c * 16, 16)] = acc
                return wk

            lax.fori_loop(0, D // 16, chunk,
                          tuple(wall_v[r * K + k] for k in range(K)))

    def pair(p, carry):
        g = 2 * p
        pltpu.async_copy(tp_hbm.at[idx_v.at[pl.ds((g + 1) * G * K, G * K)]],
                         rows_v.at[1], sg1)
        pltpu.make_async_copy(tp_hbm.at[idx_v.at[pl.ds(0, G * K)]],
                              rows_v.at[0], sg0).wait()
        accum(g, 0)
        gn = jnp.minimum((g + 2) * G * K, (NB - 1) * G * K)
        pltpu.async_copy(tp_hbm.at[idx_v.at[pl.ds(gn, G * K)]],
                         rows_v.at[0], sg0)
        pltpu.make_async_copy(tp_hbm.at[idx_v.at[pl.ds(0, G * K)]],
                              rows_v.at[1], sg1).wait()
        accum(g + 1, 1)
        return carry

    lax.fori_loop(0, NB // 2, pair, 0)
    pltpu.make_async_copy(tp_hbm.at[idx_v.at[pl.ds(0, G * K)]],
                          rows_v.at[0], sg0).wait()
    pltpu.sync_copy(acc_v, out_hbm.at[pl.ds(base, RPW)])


def _sc_gather(top_idx, top_w, txt_proj):
    mesh = plsc.VectorSubcoreMesh(core_axis_name="c", subcore_axis_name="s")
    fn = functools.partial(
        pl.kernel,
        mesh=mesh,
        out_type=jax.ShapeDtypeStruct((B, D), jnp.float32),
        scratch_types=[
            pltpu.VMEM((RPW * K,), jnp.int32),
            pltpu.VMEM((RPW * K, 16), jnp.float32),
            pltpu.VMEM((2, G * K, D), jnp.float32),
            pltpu.VMEM((RPW, D), jnp.float32),
            pltpu.SemaphoreType.DMA,
            pltpu.SemaphoreType.DMA,
        ],
    )(_gather_body)
    return fn(top_idx.reshape(B * K), top_w.reshape(B * K, 16), txt_proj)


def _final_body(ip_ref, g_ref, wfinal_ref, out_ref):
    enh = ip_ref[...] * g_ref[...]
    out = _mm(enh, wfinal_ref[...], ((1,), (1,)))
    out_ref[...] = jnp.maximum(out, 0.0)


def _final(img_proj, g, W_final):
    return pl.pallas_call(
        _final_body,
        out_shape=jax.ShapeDtypeStruct((B, D), jnp.float32),
    )(img_proj, g, W_final)


def kernel(image_features, text_features, W_img, W_txt, W_final):
    img_proj = _proj(image_features, W_img)
    txt_proj, top_w, top_idx = _topk(img_proj, text_features, W_txt)
    g = _sc_gather(top_idx, top_w, txt_proj)
    return _final(img_proj, g, W_final)


# speedup vs baseline: 3.1022x; 1.0408x over previous
import functools

import jax
import jax.numpy as jnp
from jax import lax
from jax.experimental import pallas as pl
from jax.experimental.pallas import tpu as pltpu
from jax.experimental.pallas import tpu_sc as plsc

B = 1024
N_TEXT = 16384
D = 1024
K = 8

_BF = jnp.bfloat16


def _mm(a, b, dims):
    return lax.dot_general(a.astype(_BF), b.astype(_BF), (dims, ((), ())),
                           preferred_element_type=jnp.float32)


def _proj_body(img_ref, wimg_ref, ip_ref, ipb_ref):
    ip = _mm(img_ref[...], wimg_ref[...], ((1,), (1,)))
    ip_ref[...] = ip
    ipb_ref[...] = ip.astype(_BF)


def _proj(image_features, W_img):
    return pl.pallas_call(
        _proj_body,
        out_shape=[
            jax.ShapeDtypeStruct((B, D), jnp.float32),
            jax.ShapeDtypeStruct((B, D), _BF),
        ],
    )(image_features, W_img)


B_BLK = 512
N_BLK = 2048
NI = B // B_BLK
NJ = N_TEXT // N_BLK


def _topk_body(text_ref, wtxt_ref, ipb_ref, tp_ref, w_ref, idx_ref,
               cv_ref, ci_ref):
    j = pl.program_id(0)
    i = pl.program_id(1)
    rows = pl.ds(i * B_BLK, B_BLK)

    @pl.when(i == 0)
    def _tp():
        tp_ref[...] = _mm(text_ref[...], wtxt_ref[...], ((1,), (1,)))

    s = lax.dot_general(ipb_ref[...], tp_ref[...].astype(_BF),
                        (((1,), (1,)), ((), ())),
                        preferred_element_type=jnp.float32)

    colf = lax.broadcasted_iota(jnp.int32, (B_BLK, N_BLK), 1).astype(
        jnp.float32)
    k8 = lax.broadcasted_iota(jnp.int32, (B_BLK, K), 1)

    bv = jnp.full((B_BLK, K), -jnp.inf, jnp.float32)
    bi = jnp.zeros((B_BLK, K), jnp.int32)
    for t in range(K):
        m = jnp.max(s, axis=1)
        hit = s >= m[:, None]
        af = jnp.min(jnp.where(hit, colf, jnp.float32(N_BLK)), axis=1)
        bv = jnp.where(k8 == t, m[:, None], bv)
        bi = jnp.where(k8 == t, af.astype(jnp.int32)[:, None] + j * N_BLK,
                       bi)
        s = jnp.where(hit, -jnp.inf, s)

    col128 = lax.broadcasted_iota(jnp.int32, (B_BLK, 128), 1)
    in_slot = (col128 >= j * K) & (col128 < j * K + K)
    cur_v = jnp.where(j == 0, jnp.full((B_BLK, 128), -jnp.inf, jnp.float32),
                      cv_ref[rows, :])
    cur_i = jnp.where(j == 0, jnp.zeros((B_BLK, 128), jnp.int32),
                      ci_ref[rows, :])
    cv_ref[rows, :] = jnp.where(in_slot, jnp.tile(bv, (1, 128 // K)), cur_v)
    ci_ref[rows, :] = jnp.where(in_slot, jnp.tile(bi, (1, 128 // K)), cur_i)

    @pl.when(j == NJ - 1)
    def _emit():
        cv = cv_ref[rows, :]
        ci = ci_ref[rows, :]
        colc = lax.broadcasted_iota(jnp.int32, (B_BLK, 128), 1).astype(
            jnp.float32)
        nv = jnp.zeros((B_BLK, K), jnp.float32)
        ni = jnp.zeros((B_BLK, K), jnp.int32)
        for t in range(K):
            m = jnp.max(cv, axis=1)
            hit = cv >= m[:, None]
            af = jnp.min(jnp.where(hit, colc, jnp.float32(128)), axis=1)
            a = jnp.sum(jnp.where(colc == af[:, None], ci, 0), axis=1)
            nv = jnp.where(k8 == t, m[:, None], nv)
            ni = jnp.where(k8 == t, a[:, None], ni)
            cv = jnp.where(hit, -jnp.inf, cv)
        e = jnp.exp(nv - nv[:, 0:1])
        w = e / jnp.sum(e, axis=1, keepdims=True)
        w_ref[...] = jnp.broadcast_to(w[:, :, None], (B_BLK, K, 16))
        idx_ref[...] = ni


def _topk(ip_bf16, text_features, W_txt):
    return pl.pallas_call(
        _topk_body,
        grid=(NJ, NI),
        in_specs=[
            pl.BlockSpec((N_BLK, D), lambda j, i: (j, 0)),
            pl.BlockSpec((D, D), lambda j, i: (0, 0)),
            pl.BlockSpec((B_BLK, D), lambda j, i: (i, 0)),
        ],
        out_specs=[
            pl.BlockSpec((N_BLK, D), lambda j, i: (j, 0)),
            pl.BlockSpec((B_BLK, K, 16), lambda j, i: (i, 0, 0)),
            pl.BlockSpec((B_BLK, K), lambda j, i: (i, 0)),
        ],
        out_shape=[
            jax.ShapeDtypeStruct((N_TEXT, D), jnp.float32),
            jax.ShapeDtypeStruct((B, K, 16), jnp.float32),
            jax.ShapeDtypeStruct((B, K), jnp.int32),
        ],
        scratch_shapes=[
            pltpu.VMEM((B, 128), jnp.float32),
            pltpu.VMEM((B, 128), jnp.int32),
        ],
        compiler_params=pltpu.CompilerParams(
            dimension_semantics=("arbitrary", "arbitrary")),
    )(text_features, W_txt, ip_bf16)


_NC = 2
_NS = 16
_NW = _NC * _NS
RPW = B // _NW


G = 2
NB = RPW // G


def _gather_body(idx_hbm, w_hbm, tp_hbm, out_hbm, idx_v, wall_v, rows_v,
                 acc_v, sg0, sg1):
    wid = lax.axis_index("s") * _NC + lax.axis_index("c")
    base = wid * RPW
    pltpu.sync_copy(idx_hbm.at[pl.ds(base * K, RPW * K)], idx_v)
    pltpu.sync_copy(w_hbm.at[pl.ds(base * K, RPW * K)], wall_v)
    pltpu.async_copy(tp_hbm.at[idx_v.at[pl.ds(0, G * K)]], rows_v.at[0], sg0)

    def accum(g, b):
        for rr in range(G):
            r = g * G + rr

            def chunk(c, wk):
                acc = wk[0] * rows_v[b, rr * K, pl.ds(c * 16, 16)]
                for k in range(1, K):
                    acc = acc + wk[k] * rows_v[b, rr * K + k,
                                               pl.ds(c * 16, 16)]
                acc_v[r, pl.ds(c * 16, 16)] = acc
                return wk

            lax.fori_loop(0, D // 16, chunk,
                          tuple(wall_v[r * K + k] for k in range(K)))

    def pair(p, carry):
        g = 2 * p
        pltpu.async_copy(tp_hbm.at[idx_v.at[pl.ds((g + 1) * G * K, G * K)]],
                         rows_v.at[1], sg1)
        pltpu.make_async_copy(tp_hbm.at[idx_v.at[pl.ds(0, G * K)]],
                              rows_v.at[0], sg0).wait()
        accum(g, 0)
        gn = jnp.minimum((g + 2) * G * K, (NB - 1) * G * K)
        pltpu.async_copy(tp_hbm.at[idx_v.at[pl.ds(gn, G * K)]],
                         rows_v.at[0], sg0)
        pltpu.make_async_copy(tp_hbm.at[idx_v.at[pl.ds(0, G * K)]],
                              rows_v.at[1], sg1).wait()
        accum(g + 1, 1)
        return carry

    lax.fori_loop(0, NB // 2, pair, 0)
    pltpu.make_async_copy(tp_hbm.at[idx_v.at[pl.ds(0, G * K)]],
                          rows_v.at[0], sg0).wait()
    pltpu.sync_copy(acc_v, out_hbm.at[pl.ds(base, RPW)])


def _sc_gather(top_idx, top_w, txt_proj):
    mesh = plsc.VectorSubcoreMesh(core_axis_name="c", subcore_axis_name="s")
    fn = functools.partial(
        pl.kernel,
        mesh=mesh,
        out_type=jax.ShapeDtypeStruct((B, D), jnp.float32),
        scratch_types=[
            pltpu.VMEM((RPW * K,), jnp.int32),
            pltpu.VMEM((RPW * K, 16), jnp.float32),
            pltpu.VMEM((2, G * K, D), jnp.float32),
            pltpu.VMEM((RPW, D), jnp.float32),
            pltpu.SemaphoreType.DMA,
            pltpu.SemaphoreType.DMA,
        ],
    )(_gather_body)
    return fn(top_idx.reshape(B * K), top_w.reshape(B * K, 16), txt_proj)


def _final_body(ip_ref, g_ref, wfinal_ref, out_ref):
    enh = ip_ref[...] * g_ref[...]
    out = _mm(enh, wfinal_ref[...], ((1,), (1,)))
    out_ref[...] = jnp.maximum(out, 0.0)


def _final(img_proj, g, W_final):
    return pl.pallas_call(
        _final_body,
        out_shape=jax.ShapeDtypeStruct((B, D), jnp.float32),
    )(img_proj, g, W_final)


def kernel(image_features, text_features, W_img, W_txt, W_final):
    img_proj, ip_bf16 = _proj(image_features, W_img)
    txt_proj, top_w, top_idx = _topk(ip_bf16, text_features, W_txt)
    g = _sc_gather(top_idx, top_w, txt_proj)
    return _final(img_proj, g, W_final)


# speedup vs baseline: 3.1042x; 1.0007x over previous
import functools

import jax
import jax.numpy as jnp
from jax import lax
from jax.experimental import pallas as pl
from jax.experimental.pallas import tpu as pltpu
from jax.experimental.pallas import tpu_sc as plsc

B = 1024
N_TEXT = 16384
D = 1024
K = 8

_BF = jnp.bfloat16


def _mm(a, b, dims):
    return lax.dot_general(a.astype(_BF), b.astype(_BF), (dims, ((), ())),
                           preferred_element_type=jnp.float32)


def _proj_body(img_ref, wimg_ref, ip_ref, ipb_ref):
    ip = _mm(img_ref[...], wimg_ref[...], ((1,), (1,)))
    ip_ref[...] = ip
    ipb_ref[...] = ip.astype(_BF)


def _proj(image_features, W_img):
    return pl.pallas_call(
        _proj_body,
        out_shape=[
            jax.ShapeDtypeStruct((B, D), jnp.float32),
            jax.ShapeDtypeStruct((B, D), _BF),
        ],
    )(image_features, W_img)


B_BLK = 512
N_BLK = 2048
NI = B // B_BLK
NJ = N_TEXT // N_BLK


def _topk_body(text_ref, wtxt_ref, ipb_ref, tp_ref, w_ref, idx_ref,
               cv_ref, ci_ref):
    j = pl.program_id(0)
    i = pl.program_id(1)
    rows = pl.ds(i * B_BLK, B_BLK)

    @pl.when(i == 0)
    def _tp():
        tp_ref[...] = _mm(text_ref[...], wtxt_ref[...], ((1,), (1,)))

    s = lax.dot_general(ipb_ref[...], tp_ref[...].astype(_BF),
                        (((1,), (1,)), ((), ())),
                        preferred_element_type=jnp.float32)

    colf = lax.broadcasted_iota(jnp.int32, (B_BLK, N_BLK), 1).astype(
        jnp.float32)
    k8 = lax.broadcasted_iota(jnp.int32, (B_BLK, K), 1)

    bv = jnp.full((B_BLK, K), -jnp.inf, jnp.float32)
    bi = jnp.zeros((B_BLK, K), jnp.int32)
    for t in range(K):
        m = jnp.max(s, axis=1)
        hit = s >= m[:, None]
        af = jnp.min(jnp.where(hit, colf, jnp.float32(N_BLK)), axis=1)
        bv = jnp.where(k8 == t, m[:, None], bv)
        bi = jnp.where(k8 == t, af.astype(jnp.int32)[:, None] + j * N_BLK,
                       bi)
        if t < K - 1:
            s = jnp.where(hit, -jnp.inf, s)

    col128 = lax.broadcasted_iota(jnp.int32, (B_BLK, 128), 1)
    in_slot = (col128 >= j * K) & (col128 < j * K + K)
    cur_v = jnp.where(j == 0, jnp.full((B_BLK, 128), -jnp.inf, jnp.float32),
                      cv_ref[rows, :])
    cur_i = jnp.where(j == 0, jnp.zeros((B_BLK, 128), jnp.int32),
                      ci_ref[rows, :])
    cv_ref[rows, :] = jnp.where(in_slot, jnp.tile(bv, (1, 128 // K)), cur_v)
    ci_ref[rows, :] = jnp.where(in_slot, jnp.tile(bi, (1, 128 // K)), cur_i)

    @pl.when(j == NJ - 1)
    def _emit():
        cv = cv_ref[rows, :]
        ci = ci_ref[rows, :]
        colc = lax.broadcasted_iota(jnp.int32, (B_BLK, 128), 1).astype(
            jnp.float32)
        nv = jnp.zeros((B_BLK, K), jnp.float32)
        ni = jnp.zeros((B_BLK, K), jnp.int32)
        for t in range(K):
            m = jnp.max(cv, axis=1)
            hit = cv >= m[:, None]
            af = jnp.min(jnp.where(hit, colc, jnp.float32(128)), axis=1)
            a = jnp.sum(jnp.where(colc == af[:, None], ci, 0), axis=1)
            nv = jnp.where(k8 == t, m[:, None], nv)
            ni = jnp.where(k8 == t, a[:, None], ni)
            if t < K - 1:
                cv = jnp.where(hit, -jnp.inf, cv)
        e = jnp.exp(nv - nv[:, 0:1])
        w = e / jnp.sum(e, axis=1, keepdims=True)
        w_ref[...] = jnp.broadcast_to(w[:, :, None], (B_BLK, K, 16))
        idx_ref[...] = ni


def _topk(ip_bf16, text_features, W_txt):
    return pl.pallas_call(
        _topk_body,
        grid=(NJ, NI),
        in_specs=[
            pl.BlockSpec((N_BLK, D), lambda j, i: (j, 0)),
            pl.BlockSpec((D, D), lambda j, i: (0, 0)),
            pl.BlockSpec((B_BLK, D), lambda j, i: (i, 0)),
        ],
        out_specs=[
            pl.BlockSpec((N_BLK, D), lambda j, i: (j, 0)),
            pl.BlockSpec((B_BLK, K, 16), lambda j, i: (i, 0, 0)),
            pl.BlockSpec((B_BLK, K), lambda j, i: (i, 0)),
        ],
        out_shape=[
            jax.ShapeDtypeStruct((N_TEXT, D), jnp.float32),
            jax.ShapeDtypeStruct((B, K, 16), jnp.float32),
            jax.ShapeDtypeStruct((B, K), jnp.int32),
        ],
        scratch_shapes=[
            pltpu.VMEM((B, 128), jnp.float32),
            pltpu.VMEM((B, 128), jnp.int32),
        ],
        compiler_params=pltpu.CompilerParams(
            dimension_semantics=("arbitrary", "arbitrary")),
    )(text_features, W_txt, ip_bf16)


_NC = 2
_NS = 16
_NW = _NC * _NS
RPW = B // _NW


G = 4
NB = RPW // G


def _gather_body(idx_hbm, w_hbm, tp_hbm, out_hbm, idx_v, wall_v, rows_v,
                 acc_v, sg0, sg1, so0, so1):
    wid = lax.axis_index("s") * _NC + lax.axis_index("c")
    base = wid * RPW
    pltpu.sync_copy(idx_hbm.at[pl.ds(base * K, RPW * K)], idx_v)
    pltpu.sync_copy(w_hbm.at[pl.ds(base * K, RPW * K)], wall_v)
    pltpu.async_copy(tp_hbm.at[idx_v.at[pl.ds(0, G * K)]], rows_v.at[0], sg0)

    def accum(g, b):
        for rr in range(G):
            r = g * G + rr

            def chunk(c, wk):
                acc = wk[0] * rows_v[b, rr * K, pl.ds(c * 16, 16)]
                for k in range(1, K):
                    acc = acc + wk[k] * rows_v[b, rr * K + k,
                                               pl.ds(c * 16, 16)]
                acc_v[b, rr, pl.ds(c * 16, 16)] = acc
                return wk

            lax.fori_loop(0, D // 16, chunk,
                          tuple(wall_v[r * K + k] for k in range(K)))

    def do_batch(g, b, p, so):
        pltpu.make_async_copy(tp_hbm.at[idx_v.at[pl.ds(0, G * K)]],
                              rows_v.at[b], sg0 if b == 0 else sg1).wait()

        @pl.when(p > 0)
        def _wait_out():
            pltpu.make_async_copy(acc_v.at[b],
                                  out_hbm.at[pl.ds(base, G)], so).wait()

        accum(g, b)
        pltpu.async_copy(acc_v.at[b], out_hbm.at[pl.ds(base + g * G, G)], so)

    def pair(p, carry):
        g = 2 * p
        pltpu.async_copy(tp_hbm.at[idx_v.at[pl.ds((g + 1) * G * K, G * K)]],
                         rows_v.at[1], sg1)
        do_batch(g, 0, p, so0)
        gn = jnp.minimum((g + 2) * G * K, (NB - 1) * G * K)
        pltpu.async_copy(tp_hbm.at[idx_v.at[pl.ds(gn, G * K)]],
                         rows_v.at[0], sg0)
        do_batch(g + 1, 1, p, so1)
        return carry

    lax.fori_loop(0, NB // 2, pair, 0)
    pltpu.make_async_copy(tp_hbm.at[idx_v.at[pl.ds(0, G * K)]],
                          rows_v.at[0], sg0).wait()
    pltpu.make_async_copy(acc_v.at[0], out_hbm.at[pl.ds(base, G)], so0).wait()
    pltpu.make_async_copy(acc_v.at[1], out_hbm.at[pl.ds(base, G)], so1).wait()


def _sc_gather(top_idx, top_w, txt_proj):
    mesh = plsc.VectorSubcoreMesh(core_axis_name="c", subcore_axis_name="s")
    fn = functools.partial(
        pl.kernel,
        mesh=mesh,
        out_type=jax.ShapeDtypeStruct((B, D), jnp.float32),
        scratch_types=[
            pltpu.VMEM((RPW * K,), jnp.int32),
            pltpu.VMEM((RPW * K, 16), jnp.float32),
            pltpu.VMEM((2, G * K, D), jnp.float32),
            pltpu.VMEM((2, G, D), jnp.float32),
            pltpu.SemaphoreType.DMA,
            pltpu.SemaphoreType.DMA,
            pltpu.SemaphoreType.DMA,
            pltpu.SemaphoreType.DMA,
        ],
    )(_gather_body)
    return fn(top_idx.reshape(B * K), top_w.reshape(B * K, 16), txt_proj)


def _final_body(ip_ref, g_ref, wfinal_ref, out_ref):
    enh = ip_ref[...] * g_ref[...]
    out = _mm(enh, wfinal_ref[...], ((1,), (1,)))
    out_ref[...] = jnp.maximum(out, 0.0)


def _final(img_proj, g, W_final):
    return pl.pallas_call(
        _final_body,
        out_shape=jax.ShapeDtypeStruct((B, D), jnp.float32),
    )(img_proj, g, W_final)


def kernel(image_features, text_features, W_img, W_txt, W_final):
    img_proj, ip_bf16 = _proj(image_features, W_img)
    txt_proj, top_w, top_idx = _topk(ip_bf16, text_features, W_txt)
    g = _sc_gather(top_idx, top_w, txt_proj)
    return _final(img_proj, g, W_final)
